# baseline (speedup 1.0000x reference)
import jax
import jax.numpy as jnp
from jax import lax
from jax.experimental import pallas as pl
from jax.experimental.pallas import tpu as pltpu
from jax.experimental.pallas import tpu_sc as plsc

N_USERS = 100000
N_ENTITIES = 100000
N_RELATIONS = 32
EMB_DIM = 64
KGE_DIM = 64
B = 16384

_PACKW = 128
_NPACK = (N_USERS + N_ENTITIES) // 2

_RBK = 10000
_RNB = (N_USERS // 2) // _RBK


def _repack_body(ul_ref, uh_ref, el_ref, eh_ref, out_ref):
    t = pl.program_id(0)
    lo = jnp.where(t == 0, ul_ref[...], el_ref[...])
    hi = jnp.where(t == 0, uh_ref[...], eh_ref[...])
    out_ref[...] = jnp.concatenate([lo, hi], axis=1)


def _repack(user_embed, entity_embed):
    f32 = jnp.float32
    bs = lambda f: pl.BlockSpec((_RBK, EMB_DIM), f)
    return pl.pallas_call(
        _repack_body,
        grid=(2, _RNB),
        in_specs=[
            bs(lambda t, i: (i * (1 - t), 0)),
            bs(lambda t, i: ((_RNB + i) * (1 - t), 0)),
            bs(lambda t, i: (i * t, 0)),
            bs(lambda t, i: ((_RNB + i) * t, 0)),
        ],
        out_specs=pl.BlockSpec((_RBK, _PACKW),
                               lambda t, i: (t * _RNB + i, 0)),
        out_shape=jax.ShapeDtypeStruct((_NPACK, _PACKW), f32),
    )(user_embed, user_embed, entity_embed, entity_embed)


_NC = 2
_NS = 16
_NW = _NC * _NS
_TOT = 3 * B
_PER_W = _TOT // _NW
_CH = 128
_NCHUNK = _PER_W // _CH


def _sc_gather_body(chunks, tab_hbm, h_hbm, p_hbm, n_hbm, out_hbm, *sc):
    idxv = sc[0:2]
    fidx = sc[2:4]
    rows = sc[4:6]
    si = sc[6:8]
    sg = sc[8:10]
    sw = sc[10:12]
    wid = lax.axis_index("s") * _NC + lax.axis_index("c")
    segs = (h_hbm, p_hbm, n_hbm)
    _STRIDE = _NW * _CH

    def seg_ref(j):
        c = chunks[j]
        return segs[(c * _STRIDE) // B]

    def seg_off(j):
        c = chunks[j]
        return ((c * _STRIDE) % B) + wid * _CH

    def gout(j):
        return out_hbm.at[pl.ds(j * _STRIDE + wid * _CH, _CH)]

    _NCHUNK = len(chunks)

    idx_cp = [None] * (_NCHUNK + 2)
    g = [None] * _NCHUNK
    wb = [None] * _NCHUNK
    idx_cp[0] = pltpu.async_copy(seg_ref(0).at[pl.ds(seg_off(0), _CH)],
                                 idxv[0], si[0])
    idx_cp[1] = pltpu.async_copy(seg_ref(1).at[pl.ds(seg_off(1), _CH)],
                                 idxv[1], si[1])
    for i in range(_NCHUNK):
        b = i % 2
        idx_cp[i].wait()

        @pl.loop(0, _CH, step=16)
        def _fold(j, b=b):
            v = idxv[b][pl.ds(j, 16)]
            a = jnp.where(v >= N_USERS, v - N_USERS, v)
            c = jnp.where(a >= N_USERS // 2, a - N_USERS // 2, a)
            fidx[b][pl.ds(j, 16)] = jnp.where(v >= N_USERS,
                                              c + N_USERS // 2, c)

        if i + 2 < _NCHUNK:
            idx_cp[i + 2] = pltpu.async_copy(
                seg_ref(i + 2).at[pl.ds(seg_off(i + 2), _CH)], idxv[b], si[b])
        if i >= 2:
            wb[i - 2].wait()
        g[i] = pltpu.async_copy(tab_hbm.at[fidx[b]], rows[b], sg[b])
        if i >= 1:
            pb = (i - 1) % 2
            g[i - 1].wait()
            wb[i - 1] = pltpu.async_copy(rows[pb], gout(i - 1), sw[pb])
    last = _NCHUNK - 1
    lb = last % 2
    g[last].wait()
    wb[last] = pltpu.async_copy(rows[lb], gout(last), sw[lb])
    wb[last - 1].wait()
    wb[last].wait()


def _sc_gather(packed_tab, heads, pos_tails, neg_tails, chunks):
    import functools
    mesh = plsc.VectorSubcoreMesh(core_axis_name="c", subcore_axis_name="s")
    f32 = jnp.float32
    i32 = jnp.int32
    nrows = len(chunks) * _NW * _CH
    kern = pl.kernel(
        functools.partial(_sc_gather_body, chunks),
        mesh=mesh,
        compiler_params=pltpu.CompilerParams(use_tc_tiling_on_sc=False),
        out_type=jax.ShapeDtypeStruct((nrows, _PACKW), f32),
        scratch_types=(
            [pltpu.VMEM((_CH,), i32) for _ in range(4)]
            + [pltpu.VMEM((_CH, _PACKW), f32) for _ in range(2)]
            + [pltpu.SemaphoreType.DMA for _ in range(6)]
        ),
    )
    return kern(packed_tab, heads, pos_tails, neg_tails)


_T = 1024
_GRID = B // _T
_RK = N_RELATIONS * KGE_DIM


def _tc_body(hb, pb, nb, meta, w2p, rel_flat, out_ref):
    f32 = jnp.float32
    m = meta[...]
    rel = lax.shift_right_logical(m, 3)

    def half(blk, bit):
        odd = (lax.shift_right_logical(m, bit) & 1) == 1
        return jnp.where(odd, blk[...][:, EMB_DIM:], blk[...][:, :EMB_DIM])

    h = half(hb, 2)
    p = half(pb, 1)
    n = half(nb, 0)
    dp = h - p
    dn = h - n
    xs = jnp.concatenate([dp, dn], axis=0).astype(jnp.bfloat16)
    y = lax.dot_general(xs, w2p[...], (((1,), (0,)), ((), ())),
                        preferred_element_type=f32,
                        precision=lax.Precision.DEFAULT)
    t = y + rel_flat[...]
    t2 = (t * t).astype(jnp.bfloat16)
    sel = (lax.broadcasted_iota(jnp.int32, (_RK, N_RELATIONS), 0) // KGE_DIM
           == lax.broadcasted_iota(jnp.int32, (_RK, N_RELATIONS), 1)
           ).astype(jnp.bfloat16)
    s_all = lax.dot_general(t2, sel, (((1,), (0,)), ((), ())),
                            preferred_element_type=f32,
                            precision=lax.Precision.DEFAULT)
    diff = s_all[:_T] - s_all[_T:]
    oh = (rel == lax.broadcasted_iota(jnp.int32, (_T, N_RELATIONS), 1)
          ).astype(f32)
    d = jnp.sum(diff * oh, axis=1)
    part = jnp.sum(jnp.maximum(d, 0.0) + jnp.log1p(jnp.exp(-jnp.abs(d))))

    @pl.when(pl.program_id(0) == 0)
    def _init():
        out_ref[0, 0] = 0.0

    out_ref[0, 0] += part * (1.0 / B)


def _tc_loss(rows, meta, w2p, rel_flat, interpret=False):
    f32 = jnp.float32
    nb = meta.shape[0] // _T
    row_spec = lambda seg: pl.BlockSpec((_T, _PACKW),
                                        lambda i, s=seg: (s * nb + i, 0))
    out = pl.pallas_call(
        _tc_body,
        grid=(nb,),
        in_specs=[
            row_spec(0), row_spec(1), row_spec(2),
            pl.BlockSpec((_T, 1), lambda i: (i, 0)),
            pl.BlockSpec((EMB_DIM, _RK), lambda i: (0, 0)),
            pl.BlockSpec((1, _RK), lambda i: (0, 0)),
        ],
        out_specs=pl.BlockSpec(memory_space=pltpu.SMEM),
        out_shape=jax.ShapeDtypeStruct((1, 1), f32),
        interpret=interpret,
    )(rows, rows, rows, meta, w2p, rel_flat)
    return out[0, 0]


def kernel(user_embed, entity_embed, relation_embed, trans_W,
           heads, relations, pos_tails, neg_tails):
    i32 = jnp.int32
    heads = heads.astype(i32)
    pos_tails = pos_tails.astype(i32)
    neg_tails = neg_tails.astype(i32)
    relations = relations.astype(i32)

    packed_tab = _repack(user_embed, entity_embed)
    rows_q = [_sc_gather(packed_tab, heads, pos_tails, neg_tails,
                         chunks=(q, 4 + q, 8 + q)) for q in range(4)]

    hbit = (heads // (N_USERS // 2)) & 1
    pbit = (pos_tails // (N_USERS // 2)) & 1
    nbit = (neg_tails // (N_USERS // 2)) & 1
    meta = ((relations << 3) | (hbit << 2) | (pbit << 1) | nbit
            ).reshape(B, 1)
    w2p = jnp.transpose(trans_W, (1, 0, 2)).reshape(EMB_DIM, _RK
                                                    ).astype(jnp.bfloat16)
    rel_flat = relation_embed.reshape(1, _RK)
    q4 = B // 4
    ls = [_tc_loss(rows_q[q], meta[q * q4:(q + 1) * q4], w2p, rel_flat)
          for q in range(4)]
    return ls[0] + ls[1] + ls[2] + ls[3]


# speedup vs baseline: 1.0383x; 1.0383x over previous
import jax
import jax.numpy as jnp
from jax import lax
from jax.experimental import pallas as pl
from jax.experimental.pallas import tpu as pltpu
from jax.experimental.pallas import tpu_sc as plsc

N_USERS = 100000
N_ENTITIES = 100000
N_RELATIONS = 32
EMB_DIM = 64
KGE_DIM = 64
B = 16384

_PACKW = 128
_NPACK = (N_USERS + N_ENTITIES) // 2

_RBK = 10000
_RNB = (N_USERS // 2) // _RBK


def _repack_body(ul_ref, uh_ref, el_ref, eh_ref, out_ref):
    t = pl.program_id(0)
    lo = jnp.where(t == 0, ul_ref[...], el_ref[...])
    hi = jnp.where(t == 0, uh_ref[...], eh_ref[...])
    out_ref[...] = jnp.concatenate([lo, hi], axis=1)


def _repack(user_embed, entity_embed):
    f32 = jnp.float32
    bs = lambda f: pl.BlockSpec((_RBK, EMB_DIM), f)
    return pl.pallas_call(
        _repack_body,
        grid=(2, _RNB),
        in_specs=[
            bs(lambda t, i: (i * (1 - t), 0)),
            bs(lambda t, i: ((_RNB + i) * (1 - t), 0)),
            bs(lambda t, i: (i * t, 0)),
            bs(lambda t, i: ((_RNB + i) * t, 0)),
        ],
        out_specs=pl.BlockSpec((_RBK, _PACKW),
                               lambda t, i: (t * _RNB + i, 0)),
        out_shape=jax.ShapeDtypeStruct((_NPACK, _PACKW), f32),
    )(user_embed, user_embed, entity_embed, entity_embed)


_NC = 2
_NS = 16
_NW = _NC * _NS
_TOT = 3 * B
_PER_W = _TOT // _NW
_CH = 128
_NCHUNK = _PER_W // _CH


def _sc_gather_body(chunks, tab_hbm, h_hbm, p_hbm, n_hbm, out_hbm, *sc):
    idxv = sc[0:2]
    fidx = sc[2:4]
    rows = sc[4:6]
    si = sc[6:8]
    sg = sc[8:10]
    sw = sc[10:12]
    wid = lax.axis_index("s") * _NC + lax.axis_index("c")
    segs = (h_hbm, p_hbm, n_hbm)
    _STRIDE = _NW * _CH

    def seg_ref(j):
        c = chunks[j]
        return segs[(c * _STRIDE) // B]

    def seg_off(j):
        c = chunks[j]
        return ((c * _STRIDE) % B) + wid * _CH

    def gout(j):
        return out_hbm.at[pl.ds(j * _STRIDE + wid * _CH, _CH)]

    _NCHUNK = len(chunks)

    idx_cp = [None] * (_NCHUNK + 2)
    g = [None] * _NCHUNK
    wb = [None] * _NCHUNK
    idx_cp[0] = pltpu.async_copy(seg_ref(0).at[pl.ds(seg_off(0), _CH)],
                                 idxv[0], si[0])
    idx_cp[1] = pltpu.async_copy(seg_ref(1).at[pl.ds(seg_off(1), _CH)],
                                 idxv[1], si[1])
    for i in range(_NCHUNK):
        b = i % 2
        idx_cp[i].wait()

        @pl.loop(0, _CH, step=16)
        def _fold(j, b=b):
            v = idxv[b][pl.ds(j, 16)]
            a = jnp.where(v >= N_USERS, v - N_USERS, v)
            c = jnp.where(a >= N_USERS // 2, a - N_USERS // 2, a)
            fidx[b][pl.ds(j, 16)] = jnp.where(v >= N_USERS,
                                              c + N_USERS // 2, c)

        if i + 2 < _NCHUNK:
            idx_cp[i + 2] = pltpu.async_copy(
                seg_ref(i + 2).at[pl.ds(seg_off(i + 2), _CH)], idxv[b], si[b])
        if i >= 2:
            wb[i - 2].wait()
        g[i] = pltpu.async_copy(tab_hbm.at[fidx[b]], rows[b], sg[b])
        if i >= 1:
            pb = (i - 1) % 2
            g[i - 1].wait()
            wb[i - 1] = pltpu.async_copy(rows[pb], gout(i - 1), sw[pb])
    last = _NCHUNK - 1
    lb = last % 2
    g[last].wait()
    wb[last] = pltpu.async_copy(rows[lb], gout(last), sw[lb])
    wb[last - 1].wait()
    wb[last].wait()


def _sc_gather(packed_tab, heads, pos_tails, neg_tails, chunks):
    import functools
    mesh = plsc.VectorSubcoreMesh(core_axis_name="c", subcore_axis_name="s")
    f32 = jnp.float32
    i32 = jnp.int32
    nrows = len(chunks) * _NW * _CH
    kern = pl.kernel(
        functools.partial(_sc_gather_body, chunks),
        mesh=mesh,
        compiler_params=pltpu.CompilerParams(use_tc_tiling_on_sc=False),
        out_type=jax.ShapeDtypeStruct((nrows, _PACKW), f32),
        scratch_types=(
            [pltpu.VMEM((_CH,), i32) for _ in range(4)]
            + [pltpu.VMEM((_CH, _PACKW), f32) for _ in range(2)]
            + [pltpu.SemaphoreType.DMA for _ in range(6)]
        ),
    )
    return kern(packed_tab, heads, pos_tails, neg_tails)


_T = 1024
_GRID = B // _T
_RK = N_RELATIONS * KGE_DIM


def _tc_body(hb, pb, nb, meta, w2p, rel_flat, out_ref):
    f32 = jnp.float32
    m = meta[...]
    rel = lax.shift_right_logical(m, 3)

    def half(blk, bit):
        odd = (lax.shift_right_logical(m, bit) & 1) == 1
        return jnp.where(odd, blk[...][:, EMB_DIM:], blk[...][:, :EMB_DIM])

    h = half(hb, 2)
    p = half(pb, 1)
    n = half(nb, 0)
    dp = h - p
    dn = h - n
    xs = jnp.concatenate([dp, dn], axis=0).astype(jnp.bfloat16)
    y = lax.dot_general(xs, w2p[...], (((1,), (0,)), ((), ())),
                        preferred_element_type=f32,
                        precision=lax.Precision.DEFAULT)
    t = y + rel_flat[...]
    t2 = (t * t).astype(jnp.bfloat16)
    sel = (lax.broadcasted_iota(jnp.int32, (_RK, N_RELATIONS), 0) // KGE_DIM
           == lax.broadcasted_iota(jnp.int32, (_RK, N_RELATIONS), 1)
           ).astype(jnp.bfloat16)
    s_all = lax.dot_general(t2, sel, (((1,), (0,)), ((), ())),
                            preferred_element_type=f32,
                            precision=lax.Precision.DEFAULT)
    diff = s_all[:_T] - s_all[_T:]
    oh = (rel == lax.broadcasted_iota(jnp.int32, (_T, N_RELATIONS), 1)
          ).astype(f32)
    d = jnp.sum(diff * oh, axis=1)
    part = jnp.sum(jnp.maximum(d, 0.0) + jnp.log1p(jnp.exp(-jnp.abs(d))))

    @pl.when(pl.program_id(0) == 0)
    def _init():
        out_ref[0, 0] = 0.0

    out_ref[0, 0] += part * (1.0 / B)


def _tc_loss(rows, meta, w2p, rel_flat, interpret=False):
    f32 = jnp.float32
    nb = meta.shape[0] // _T
    row_spec = lambda seg: pl.BlockSpec((_T, _PACKW),
                                        lambda i, s=seg: (s * nb + i, 0))
    out = pl.pallas_call(
        _tc_body,
        grid=(nb,),
        in_specs=[
            row_spec(0), row_spec(1), row_spec(2),
            pl.BlockSpec((_T, 1), lambda i: (i, 0)),
            pl.BlockSpec((EMB_DIM, _RK), lambda i: (0, 0)),
            pl.BlockSpec((1, _RK), lambda i: (0, 0)),
        ],
        out_specs=pl.BlockSpec(memory_space=pltpu.SMEM),
        out_shape=jax.ShapeDtypeStruct((1, 1), f32),
        interpret=interpret,
    )(rows, rows, rows, meta, w2p, rel_flat)
    return out[0, 0]


def kernel(user_embed, entity_embed, relation_embed, trans_W,
           heads, relations, pos_tails, neg_tails):
    i32 = jnp.int32
    heads = heads.astype(i32)
    pos_tails = pos_tails.astype(i32)
    neg_tails = neg_tails.astype(i32)
    relations = relations.astype(i32)

    packed_tab = _repack(user_embed, entity_embed)
    rows1 = _sc_gather(packed_tab, heads, pos_tails, neg_tails,
                       chunks=(0, 1, 4, 5, 8, 9))
    rows2 = _sc_gather(packed_tab, heads, pos_tails, neg_tails,
                       chunks=(2, 3, 6, 7, 10, 11))

    hbit = (heads // (N_USERS // 2)) & 1
    pbit = (pos_tails // (N_USERS // 2)) & 1
    nbit = (neg_tails // (N_USERS // 2)) & 1
    meta = ((relations << 3) | (hbit << 2) | (pbit << 1) | nbit
            ).reshape(B, 1)
    w2p = jnp.transpose(trans_W, (1, 0, 2)).reshape(EMB_DIM, _RK
                                                    ).astype(jnp.bfloat16)
    rel_flat = relation_embed.reshape(1, _RK)
    l1 = _tc_loss(rows1, meta[:B // 2], w2p, rel_flat)
    l2 = _tc_loss(rows2, meta[B // 2:], w2p, rel_flat)
    return l1 + l2


# speedup vs baseline: 1.0409x; 1.0025x over previous
import functools

import jax
import jax.numpy as jnp
from jax import lax
from jax.experimental import pallas as pl
from jax.experimental.pallas import tpu as pltpu
from jax.experimental.pallas import tpu_sc as plsc

N_USERS = 100000
N_ENTITIES = 100000
N_RELATIONS = 32
EMB_DIM = 64
KGE_DIM = 64
B = 16384

_PACKW = 128
_NPACK = (N_USERS + N_ENTITIES) // 2

_RBK = 10000
_RNB = (N_USERS // 2) // _RBK


def _repack_body(ul_ref, uh_ref, el_ref, eh_ref, out_ref):
    t = pl.program_id(0)
    lo = jnp.where(t == 0, ul_ref[...], el_ref[...])
    hi = jnp.where(t == 0, uh_ref[...], eh_ref[...])
    out_ref[...] = jnp.concatenate([lo, hi], axis=1)


def _repack(user_embed, entity_embed):
    f32 = jnp.float32
    bs = lambda f: pl.BlockSpec((_RBK, EMB_DIM), f)
    return pl.pallas_call(
        _repack_body,
        grid=(2, _RNB),
        in_specs=[
            bs(lambda t, i: (i * (1 - t), 0)),
            bs(lambda t, i: ((_RNB + i) * (1 - t), 0)),
            bs(lambda t, i: (i * t, 0)),
            bs(lambda t, i: ((_RNB + i) * t, 0)),
        ],
        out_specs=pl.BlockSpec((_RBK, _PACKW),
                               lambda t, i: (t * _RNB + i, 0)),
        out_shape=jax.ShapeDtypeStruct((_NPACK, _PACKW), f32),
    )(user_embed, user_embed, entity_embed, entity_embed)


_NC = 2
_NS = 16
_NW = _NC * _NS
_TOT = 3 * B
_PER_W = _TOT // _NW
_CH = 128
_NCHUNK = _PER_W // _CH


def _sc_gather_body(chunks, tab_hbm, h_hbm, p_hbm, n_hbm, out_hbm, *sc):
    idxv = sc[0:2]
    fidx = sc[2:4]
    rows = sc[4:6]
    si = sc[6:8]
    sg = sc[8:10]
    sw = sc[10:12]
    wid = lax.axis_index("s") * _NC + lax.axis_index("c")
    segs = (h_hbm, p_hbm, n_hbm)
    _STRIDE = _NW * _CH

    def seg_ref(j):
        c = chunks[j]
        return segs[(c * _STRIDE) // B]

    def seg_off(j):
        c = chunks[j]
        return ((c * _STRIDE) % B) + wid * _CH

    def gout(j):
        return out_hbm.at[pl.ds(j * _STRIDE + wid * _CH, _CH)]

    _NCHUNK = len(chunks)

    idx_cp = [None] * (_NCHUNK + 2)
    g = [None] * _NCHUNK
    wb = [None] * _NCHUNK
    idx_cp[0] = pltpu.async_copy(seg_ref(0).at[pl.ds(seg_off(0), _CH)],
                                 idxv[0], si[0])
    idx_cp[1] = pltpu.async_copy(seg_ref(1).at[pl.ds(seg_off(1), _CH)],
                                 idxv[1], si[1])
    for i in range(_NCHUNK):
        b = i % 2
        idx_cp[i].wait()

        @pl.loop(0, _CH, step=16)
        def _fold(j, b=b):
            v = idxv[b][pl.ds(j, 16)]
            a = jnp.where(v >= N_USERS, v - N_USERS, v)
            c = jnp.where(a >= N_USERS // 2, a - N_USERS // 2, a)
            fidx[b][pl.ds(j, 16)] = jnp.where(v >= N_USERS,
                                              c + N_USERS // 2, c)

        if i + 2 < _NCHUNK:
            idx_cp[i + 2] = pltpu.async_copy(
                seg_ref(i + 2).at[pl.ds(seg_off(i + 2), _CH)], idxv[b], si[b])
        if i >= 2:
            wb[i - 2].wait()
        g[i] = pltpu.async_copy(tab_hbm.at[fidx[b]], rows[b], sg[b])
        if i >= 1:
            pb = (i - 1) % 2
            g[i - 1].wait()
            wb[i - 1] = pltpu.async_copy(rows[pb], gout(i - 1), sw[pb])
    last = _NCHUNK - 1
    lb = last % 2
    g[last].wait()
    wb[last] = pltpu.async_copy(rows[lb], gout(last), sw[lb])
    wb[last - 1].wait()
    wb[last].wait()


def _sc_gather(packed_tab, heads, pos_tails, neg_tails, chunks):
    mesh = plsc.VectorSubcoreMesh(core_axis_name="c", subcore_axis_name="s")
    f32 = jnp.float32
    i32 = jnp.int32
    nrows = len(chunks) * _NW * _CH
    kern = pl.kernel(
        functools.partial(_sc_gather_body, chunks),
        mesh=mesh,
        compiler_params=pltpu.CompilerParams(use_tc_tiling_on_sc=False),
        out_type=jax.ShapeDtypeStruct((nrows, _PACKW), f32),
        scratch_types=(
            [pltpu.VMEM((_CH,), i32) for _ in range(4)]
            + [pltpu.VMEM((_CH, _PACKW), f32) for _ in range(2)]
            + [pltpu.SemaphoreType.DMA for _ in range(6)]
        ),
    )
    return kern(packed_tab, heads, pos_tails, neg_tails)


_T = 1024
_GRID = B // _T
_RK = N_RELATIONS * KGE_DIM


def _tc_body(hb, pb, nb, meta, w2p, rel_flat, out_ref):
    f32 = jnp.float32
    m = meta[...]
    rel = lax.shift_right_logical(m, 3)

    def half(blk, bit):
        odd = (lax.shift_right_logical(m, bit) & 1) == 1
        return jnp.where(odd, blk[...][:, EMB_DIM:], blk[...][:, :EMB_DIM])

    h = half(hb, 2)
    p = half(pb, 1)
    n = half(nb, 0)
    dp = h - p
    dn = h - n
    xs = jnp.concatenate([dp, dn], axis=0).astype(jnp.bfloat16)
    y = lax.dot_general(xs, w2p[...], (((1,), (0,)), ((), ())),
                        preferred_element_type=f32,
                        precision=lax.Precision.DEFAULT)
    t = y + rel_flat[...]
    t2 = (t * t).astype(jnp.bfloat16)
    sel = (lax.broadcasted_iota(jnp.int32, (_RK, N_RELATIONS), 0) // KGE_DIM
           == lax.broadcasted_iota(jnp.int32, (_RK, N_RELATIONS), 1)
           ).astype(jnp.bfloat16)
    s_all = lax.dot_general(t2, sel, (((1,), (0,)), ((), ())),
                            preferred_element_type=f32,
                            precision=lax.Precision.DEFAULT)
    diff = s_all[:_T] - s_all[_T:]
    oh = (rel == lax.broadcasted_iota(jnp.int32, (_T, N_RELATIONS), 1)
          ).astype(f32)
    d = jnp.sum(diff * oh, axis=1)
    part = jnp.sum(jnp.maximum(d, 0.0) + jnp.log1p(jnp.exp(-jnp.abs(d))))

    @pl.when(pl.program_id(0) == 0)
    def _init():
        out_ref[0, 0] = 0.0

    out_ref[0, 0] += part * (1.0 / B)


def _tc_loss(rows, meta, w2p, rel_flat):
    f32 = jnp.float32
    nb = meta.shape[0] // _T
    row_spec = lambda seg: pl.BlockSpec((_T, _PACKW),
                                        lambda i, s=seg: (s * nb + i, 0))
    out = pl.pallas_call(
        _tc_body,
        grid=(nb,),
        in_specs=[
            row_spec(0), row_spec(1), row_spec(2),
            pl.BlockSpec((_T, 1), lambda i: (i, 0)),
            pl.BlockSpec((EMB_DIM, _RK), lambda i: (0, 0)),
            pl.BlockSpec((1, _RK), lambda i: (0, 0)),
        ],
        out_specs=pl.BlockSpec(memory_space=pltpu.SMEM),
        out_shape=jax.ShapeDtypeStruct((1, 1), f32),
    )(rows, rows, rows, meta, w2p, rel_flat)
    return out[0, 0]


def kernel(user_embed, entity_embed, relation_embed, trans_W,
           heads, relations, pos_tails, neg_tails):
    i32 = jnp.int32
    heads = heads.astype(i32)
    pos_tails = pos_tails.astype(i32)
    neg_tails = neg_tails.astype(i32)
    relations = relations.astype(i32)

    packed_tab = _repack(user_embed, entity_embed)
    rows1 = _sc_gather(packed_tab, heads, pos_tails, neg_tails,
                       chunks=(0, 1, 4, 5, 8, 9))
    rows2 = _sc_gather(packed_tab, heads, pos_tails, neg_tails,
                       chunks=(2, 3, 6, 7, 10, 11))

    hbit = (heads // (N_USERS // 2)) & 1
    pbit = (pos_tails // (N_USERS // 2)) & 1
    nbit = (neg_tails // (N_USERS // 2)) & 1
    meta = ((relations << 3) | (hbit << 2) | (pbit << 1) | nbit
            ).reshape(B, 1)
    w2p = jnp.transpose(trans_W, (1, 0, 2)).reshape(EMB_DIM, _RK
                                                    ).astype(jnp.bfloat16)
    rel_flat = relation_embed.reshape(1, _RK)
    l1 = _tc_loss(rows1, meta[:B // 2], w2p, rel_flat)
    l2 = _tc_loss(rows2, meta[B // 2:], w2p, rel_flat)
    return l1 + l2
